# Initial kernel scaffold; baseline (speedup 1.0000x reference)
import jax
import jax.numpy as jnp
from jax.experimental import pallas as pl


def kernel(tokens, W_E):
    raise NotImplementedError("write your pallas kernel here")


# speedup vs baseline: 1.5392x; 1.5392x over previous
import functools

import jax
import jax.numpy as jnp
from jax import lax
from jax.experimental import pallas as pl
from jax.experimental.pallas import tpu as pltpu
from jax.experimental.pallas import tpu_sc as plsc

D_MODEL = 1024
NC = 2
NS = 16
NW = NC * NS

B = 16384
B_PER_W = B // NW
CHUNK = 64
NCHUNKS = B_PER_W // CHUNK

_mesh = plsc.VectorSubcoreMesh(core_axis_name="c", subcore_axis_name="s")


@functools.partial(
    pl.kernel,
    mesh=_mesh,
    out_type=jax.ShapeDtypeStruct((B, D_MODEL), jnp.float32),
    scratch_types=[
        pltpu.VMEM((B_PER_W,), jnp.int32),
        pltpu.VMEM((CHUNK, D_MODEL), jnp.float32),
        pltpu.SemaphoreType.DMA,
    ],
)
def _embed_gather(tok_hbm, table_hbm, out_hbm, idx_v, buf, sem):
    wid = lax.axis_index("s") * NC + lax.axis_index("c")
    base = wid * B_PER_W
    pltpu.sync_copy(tok_hbm.at[pl.ds(base, B_PER_W)], idx_v)
    for c in range(NCHUNKS):
        pltpu.async_copy(
            table_hbm.at[idx_v.at[pl.ds(c * CHUNK, CHUNK)]], buf, sem
        ).wait()
        pltpu.sync_copy(buf, out_hbm.at[pl.ds(base + c * CHUNK, CHUNK)])


def kernel(tokens, W_E):
    bsz, seq = tokens.shape
    tok = tokens.reshape(-1).astype(jnp.int32)
    out = _embed_gather(tok, W_E)
    return out.reshape(bsz, seq, D_MODEL)


# speedup vs baseline: 1.6378x; 1.0640x over previous
import functools

import jax
import jax.numpy as jnp
from jax import lax
from jax.experimental import pallas as pl
from jax.experimental.pallas import tpu as pltpu
from jax.experimental.pallas import tpu_sc as plsc

D_MODEL = 1024
NC = 2
NS = 16
NW = NC * NS

B = 16384
B_PER_W = B // NW
CHUNK = 32
NCHUNKS = B_PER_W // CHUNK

_mesh = plsc.VectorSubcoreMesh(core_axis_name="c", subcore_axis_name="s")


@functools.partial(
    pl.kernel,
    mesh=_mesh,
    out_type=jax.ShapeDtypeStruct((B, D_MODEL), jnp.float32),
    scratch_types=[
        pltpu.VMEM((B_PER_W,), jnp.int32),
        pltpu.VMEM((2, CHUNK, D_MODEL), jnp.float32),
        pltpu.SemaphoreType.DMA,
        pltpu.SemaphoreType.DMA,
        pltpu.SemaphoreType.DMA,
        pltpu.SemaphoreType.DMA,
    ],
)
def _embed_gather(tok_hbm, table_hbm, out_hbm, idx_v, buf, si0, si1, so0, so1):
    wid = lax.axis_index("s") * NC + lax.axis_index("c")
    base = wid * B_PER_W
    pltpu.sync_copy(tok_hbm.at[pl.ds(base, B_PER_W)], idx_v)
    sin, sout = (si0, si1), (so0, so1)

    def gather(c):
        return pltpu.async_copy(
            table_hbm.at[idx_v.at[pl.ds(c * CHUNK, CHUNK)]],
            buf.at[c % 2], sin[c % 2])

    def put(c):
        return pltpu.async_copy(
            buf.at[c % 2], out_hbm.at[pl.ds(base + c * CHUNK, CHUNK)],
            sout[c % 2])

    g = [None] * NCHUNKS
    p = [None] * NCHUNKS
    g[0] = gather(0)
    for c in range(NCHUNKS):
        if c + 1 < NCHUNKS:
            if c >= 1:
                p[c - 1].wait()
            g[c + 1] = gather(c + 1)
        g[c].wait()
        p[c] = put(c)
    p[NCHUNKS - 2].wait()
    p[NCHUNKS - 1].wait()


def kernel(tokens, W_E):
    bsz, seq = tokens.shape
    tok = tokens.reshape(-1).astype(jnp.int32)
    out = _embed_gather(tok, W_E)
    return out.reshape(bsz, seq, D_MODEL)
